# Initial kernel scaffold; baseline (speedup 1.0000x reference)
import jax
import jax.numpy as jnp
from jax.experimental import pallas as pl


def kernel(x, hidden, E, W, U, b, rb):
    raise NotImplementedError("write your pallas kernel here")


# speedup vs baseline: 5.3979x; 5.3979x over previous
import functools

import jax
import jax.numpy as jnp
from jax import lax
from jax.experimental import pallas as pl
from jax.experimental.pallas import tpu as pltpu
from jax.experimental.pallas import tpu_sc as plsc

VOCAB = 100000
EMBED = 128
UNITS = 128
B = 1024
T = 200
N = B * T
H3 = 3 * UNITS

_NC = 2
_NS = 16
_NW = _NC * _NS
_PER_W = N // _NW
_CHUNK = 640
_NCHUNK = _PER_W // _CHUNK
_SUB = 128
_NSUB = _CHUNK // _SUB


def _make_gather():
    mesh = plsc.VectorSubcoreMesh(core_axis_name="c", subcore_axis_name="s")

    @functools.partial(
        pl.kernel,
        mesh=mesh,
        out_type=jax.ShapeDtypeStruct((N, EMBED), jnp.float32),
        scratch_types=[
            pltpu.VMEM((_CHUNK,), jnp.int32),
            pltpu.VMEM((_CHUNK, EMBED), jnp.float32),
            pltpu.SemaphoreType.DMA,
        ],
    )
    def gather_k(idx_hbm, table_hbm, out_hbm, idx_v, rows_v, sem):
        wid = lax.axis_index("s") * _NC + lax.axis_index("c")
        base = wid * _PER_W

        def body(c, carry):
            off = base + c * _CHUNK
            pltpu.sync_copy(idx_hbm.at[pl.ds(off, _CHUNK)], idx_v)
            cps = [
                pltpu.async_copy(
                    table_hbm.at[idx_v.at[pl.ds(j * _SUB, _SUB)]],
                    rows_v.at[pl.ds(j * _SUB, _SUB)],
                    sem,
                )
                for j in range(_NSUB)
            ]
            for cp in cps:
                cp.wait()
            pltpu.sync_copy(rows_v, out_hbm.at[pl.ds(off, _CHUNK)])
            return carry

        lax.fori_loop(0, _NCHUNK, body, 0)

    return gather_k


_BB = 512
_TT = 8
_NB = B // _BB
_NT = T // _TT


def _gru_body(emb_ref, h0_ref, w_ref, u_ref, b_ref, rb_ref,
              out_ref, state_ref, h_scr):
    it = pl.program_id(1)

    @pl.when(it == 0)
    def _():
        h_scr[...] = h0_ref[...]

    w = w_ref[...]
    u = u_ref[...]
    bias = b_ref[...]
    rbias = rb_ref[...]
    emb = emb_ref[...].reshape(_BB * _TT, EMBED)
    xm = jnp.dot(emb, w, preferred_element_type=jnp.float32)
    xm = xm.reshape(_BB, _TT, H3) + bias[:, None, :]

    h = h_scr[...]
    for k in range(_TT):
        x_t = xm[:, k, :]
        hm = jnp.dot(h, u, preferred_element_type=jnp.float32) + rbias
        z = jax.nn.sigmoid(x_t[:, :UNITS] + hm[:, :UNITS])
        r = jax.nn.sigmoid(x_t[:, UNITS:2 * UNITS] + hm[:, UNITS:2 * UNITS])
        hh = jnp.tanh(x_t[:, 2 * UNITS:] + r * hm[:, 2 * UNITS:])
        h = z * h + (1.0 - z) * hh
        out_ref[:, k, :] = h
    h_scr[...] = h
    state_ref[...] = h


def _gru_call(emb, hidden, W, U, b2, rb2):
    return pl.pallas_call(
        _gru_body,
        grid=(_NB, _NT),
        in_specs=[
            pl.BlockSpec((_BB, _TT, EMBED), lambda ib, it: (ib, it, 0)),
            pl.BlockSpec((_BB, UNITS), lambda ib, it: (ib, 0)),
            pl.BlockSpec((EMBED, H3), lambda ib, it: (0, 0)),
            pl.BlockSpec((UNITS, H3), lambda ib, it: (0, 0)),
            pl.BlockSpec((1, H3), lambda ib, it: (0, 0)),
            pl.BlockSpec((1, H3), lambda ib, it: (0, 0)),
        ],
        out_specs=[
            pl.BlockSpec((_BB, _TT, UNITS), lambda ib, it: (ib, it, 0)),
            pl.BlockSpec((_BB, UNITS), lambda ib, it: (ib, 0)),
        ],
        out_shape=[
            jax.ShapeDtypeStruct((B, T, UNITS), jnp.float32),
            jax.ShapeDtypeStruct((B, UNITS), jnp.float32),
        ],
        scratch_shapes=[pltpu.VMEM((_BB, UNITS), jnp.float32)],
    )(emb, hidden, W, U, b2, rb2)


def kernel(x, hidden, E, W, U, b, rb):
    idx = x.reshape(N).astype(jnp.int32)
    emb = _make_gather()(idx, E)
    emb = emb.reshape(B, T, EMBED)
    out, state = _gru_call(emb, hidden, W, U,
                           b.reshape(1, H3), rb.reshape(1, H3))
    return (out, state)


# speedup vs baseline: 5.4397x; 1.0078x over previous
import functools

import jax
import jax.numpy as jnp
from jax import lax
from jax.experimental import pallas as pl
from jax.experimental.pallas import tpu as pltpu
from jax.experimental.pallas import tpu_sc as plsc

VOCAB = 100000
EMBED = 128
UNITS = 128
B = 1024
T = 200
N = B * T
H3 = 3 * UNITS

_NC = 2
_NS = 16
_NW = _NC * _NS
_PER_W = N // _NW
_CHUNK = 640
_NCHUNK = _PER_W // _CHUNK
_SUB = 128
_NSUB = _CHUNK // _SUB


def _make_gather():
    mesh = plsc.VectorSubcoreMesh(core_axis_name="c", subcore_axis_name="s")

    @functools.partial(
        pl.kernel,
        mesh=mesh,
        out_type=jax.ShapeDtypeStruct((N, EMBED), jnp.float32),
        scratch_types=[
            pltpu.VMEM((_CHUNK,), jnp.int32),
            pltpu.VMEM((_CHUNK, EMBED), jnp.float32),
            pltpu.SemaphoreType.DMA,
        ],
    )
    def gather_k(idx_hbm, table_hbm, out_hbm, idx_v, rows_v, sem):
        wid = lax.axis_index("s") * _NC + lax.axis_index("c")
        base = wid * _PER_W

        def body(c, carry):
            off = base + c * _CHUNK
            pltpu.sync_copy(idx_hbm.at[pl.ds(off, _CHUNK)], idx_v)
            cps = [
                pltpu.async_copy(
                    table_hbm.at[idx_v.at[pl.ds(j * _SUB, _SUB)]],
                    rows_v.at[pl.ds(j * _SUB, _SUB)],
                    sem,
                )
                for j in range(_NSUB)
            ]
            for cp in cps:
                cp.wait()
            pltpu.sync_copy(rows_v, out_hbm.at[pl.ds(off, _CHUNK)])
            return carry

        lax.fori_loop(0, _NCHUNK, body, 0)

    return gather_k


_BB = 512
_TT = 8
_NB = B // _BB
_NT = T // _TT


def _gru_body(emb_ref, h0_ref, w_ref, u_ref, b_ref, rb_ref,
              out_ref, state_ref, h_scr):
    it = pl.program_id(1)

    @pl.when(it == 0)
    def _():
        h_scr[...] = h0_ref[...]

    w = w_ref[...]
    u = u_ref[...]
    bias = b_ref[...]
    rbias = rb_ref[...]
    emb = emb_ref[...].reshape(_BB * _TT, EMBED)
    xm = jnp.dot(emb, w, preferred_element_type=jnp.float32)
    xm = xm.reshape(_BB, _TT, H3) + bias[:, None, :]

    h = h_scr[...]
    for k in range(_TT):
        x_t = xm[:, k, :]
        hm = jnp.dot(h, u, preferred_element_type=jnp.float32) + rbias
        zr = 0.5 + 0.5 * jnp.tanh(
            0.5 * (x_t[:, :2 * UNITS] + hm[:, :2 * UNITS]))
        z = zr[:, :UNITS]
        r = zr[:, UNITS:]
        hh = jnp.tanh(x_t[:, 2 * UNITS:] + r * hm[:, 2 * UNITS:])
        h = z * h + (1.0 - z) * hh
        out_ref[:, k, :] = h
    h_scr[...] = h
    state_ref[...] = h


def _gru_call(emb, hidden, W, U, b2, rb2):
    return pl.pallas_call(
        _gru_body,
        grid=(_NB, _NT),
        in_specs=[
            pl.BlockSpec((_BB, _TT, EMBED), lambda ib, it: (ib, it, 0)),
            pl.BlockSpec((_BB, UNITS), lambda ib, it: (ib, 0)),
            pl.BlockSpec((EMBED, H3), lambda ib, it: (0, 0)),
            pl.BlockSpec((UNITS, H3), lambda ib, it: (0, 0)),
            pl.BlockSpec((1, H3), lambda ib, it: (0, 0)),
            pl.BlockSpec((1, H3), lambda ib, it: (0, 0)),
        ],
        out_specs=[
            pl.BlockSpec((_BB, _TT, UNITS), lambda ib, it: (ib, it, 0)),
            pl.BlockSpec((_BB, UNITS), lambda ib, it: (ib, 0)),
        ],
        out_shape=[
            jax.ShapeDtypeStruct((B, T, UNITS), jnp.float32),
            jax.ShapeDtypeStruct((B, UNITS), jnp.float32),
        ],
        scratch_shapes=[pltpu.VMEM((_BB, UNITS), jnp.float32)],
    )(emb, hidden, W, U, b2, rb2)


def kernel(x, hidden, E, W, U, b, rb):
    idx = x.reshape(N).astype(jnp.int32)
    emb = _make_gather()(idx, E)
    emb = emb.reshape(B, T, EMBED)
    out, state = _gru_call(emb, hidden, W, U,
                           b.reshape(1, H3), rb.reshape(1, H3))
    return (out, state)
